# baseline (speedup 1.0000x reference)
import functools

import jax
import jax.numpy as jnp
from jax import lax
from jax.experimental import pallas as pl
from jax.experimental.pallas import tpu as pltpu
from jax.experimental.pallas import tpu_sc as plsc

B = 16384
L = 50
D = 64
DP = 128
V = 1000000
TBLK = 1024
NTB = (V + TBLK - 1) // TBLK
VP = NTB * (TBLK // 2)
NC = 2
NS = 16
NW = NC * NS
PW = B // NW
NID = PW * L
NLANE = 16
ND = D // NLANE
CH = 128
LH = 56
NCH = PW // CH
SPP = 8
RING = SPP * L
NP = PW // SPP
HALF = RING // 2
SPLITS = ((0, 104), (104, 96))


def _transpose_body(wt_ref, out_ref):
    x = wt_ref[...]
    eye = jnp.eye(D, dtype=jnp.float32)
    dn = (((0,), (0,)), ((), ()))
    a = lax.dot_general(x[:, : TBLK // 2], eye, dn,
                        preferred_element_type=jnp.float32)
    b = lax.dot_general(x[:, TBLK // 2 :], eye, dn,
                        preferred_element_type=jnp.float32)
    out_ref[...] = jnp.concatenate([a, b], axis=1)


def _qenc_body(idst_hbm, w_hbm, out_hbm,
               stage0, stage1, idsp_v, half_v, inv_v, ring_v, outst_v,
               semS0, semS1, semA, semB, semO):
    wid = lax.axis_index("s") * NC + lax.axis_index("c")
    sbase = wid * PW
    lane = lax.iota(jnp.int32, NLANE)
    lane50 = lane * L

    stages = (stage0, stage1)
    sems = (semS0, semS1)
    lane56 = lane * LH

    def stage_copy(c, buf, sem):
        return pltpu.async_copy(
            idst_hbm.at[:, pl.ds(sbase + CH * c, CH)], buf, sem)

    stage_copy(0, stages[0], sems[0])
    for c in range(NCH):
        buf, sem = stages[c % 2], sems[c % 2]
        if c + 1 < NCH:
            stage_copy(c + 1, stages[(c + 1) % 2], sems[(c + 1) % 2])
        pltpu.make_async_copy(
            idst_hbm.at[:, pl.ds(sbase + CH * c, CH)], buf, sem).wait()
        for k in range(CH // NLANE):
            base50 = (CH * c + NLANE * k) * L
            base56 = (CH * c + NLANE * k) * LH

            def tl(l, cnt, _k=k, _base50=base50, _base56=base56, _buf=buf):
                v = _buf[l, pl.ds(NLANE * _k, NLANE)]
                prow = (lax.shift_right_logical(v, 10) * (TBLK // 2)
                        + (v & (TBLK // 2 - 1)))
                plsc.store_scatter(idsp_v, [lane50 + (_base50 + l)], prow)
                hoffv = (lax.shift_right_logical(v, 9) & 1) * D
                plsc.store_scatter(half_v, [lane56 + (_base56 + l)], hoffv)
                return cnt + jnp.where(v != 0, 1.0, 0.0)

            cnt = lax.fori_loop(
                0, L, tl, jnp.zeros((NLANE,), jnp.float32))
            inv_v[pl.ds(CH * c + NLANE * k, NLANE)] = (
                1.0 / jnp.maximum(cnt, 1.0))

    def half_streams(p, half):
        base = pl.multiple_of(RING * p + HALF * half, 8)
        out = []
        for off, size in SPLITS:
            src = w_hbm.at[idsp_v.at[pl.ds(base + off, size)]]
            dst = ring_v.at[pl.ds(HALF * half + off, size)]
            out.append((src, dst))
        return out

    def issue(p, half, sem):
        for src, dst in half_streams(p, half):
            pltpu.async_copy(src, dst, sem)

    def drain(p, half, sem):
        for src, dst in half_streams(p, half):
            pltpu.make_async_copy(src, dst, sem).wait()

    def reduce_half(p, half, out_v):
        def one(j, carry):
            rb = HALF * half + L * j
            s = SPP * p + (SPP // 2) * half + j
            hb = LH * s
            hvecs = [half_v[pl.ds(hb + NLANE * q, NLANE)]
                     for q in range((L + NLANE - 1) // NLANE)]

            def hoff(l):
                return pl.multiple_of(hvecs[l // NLANE][l % NLANE], 8)

            accs = [ring_v[rb, pl.ds(hoff(0) + d * NLANE, NLANE)]
                    for d in range(ND)]
            for l in range(1, L):
                h = hoff(l)
                for d in range(ND):
                    accs[d] = accs[d] + ring_v[rb + l,
                                               pl.ds(h + d * NLANE, NLANE)]
            inv = plsc.load_gather(inv_v, [jnp.zeros((NLANE,), jnp.int32) + s])
            so = (SPP // 2) * half + j
            for d in range(ND):
                out_v[so, pl.ds(d * NLANE, NLANE)] = accs[d] * inv
            return carry

        lax.fori_loop(0, SPP // 2, one, 0)

    issue(0, 0, semA)
    issue(0, 1, semB)

    def out_dst(p):
        return out_hbm.at[pl.ds(sbase + SPP * p, SPP)]

    def body2(p, par):
        out_v = outst_v.at[par]

        @pl.when(p >= 2)
        def _():
            pltpu.make_async_copy(out_v, out_dst(p - 2), semO).wait()

        drain(p, 0, semA)
        reduce_half(p, 0, out_v)

        @pl.when(p + 1 < NP)
        def _():
            issue(p + 1, 0, semA)

        drain(p, 1, semB)
        reduce_half(p, 1, out_v)

        @pl.when(p + 1 < NP)
        def _():
            issue(p + 1, 1, semB)

        pltpu.async_copy(out_v, out_dst(p), semO)

    def body(g, carry):
        body2(2 * g, 0)
        body2(2 * g + 1, 1)
        return carry

    lax.fori_loop(0, NP // 2, body, 0)
    pltpu.make_async_copy(outst_v.at[0], out_dst(NP - 2), semO).wait()
    pltpu.make_async_copy(outst_v.at[1], out_dst(NP - 1), semO).wait()


@jax.jit
def _qenc(ids_t, w_t):
    wpairs = pl.pallas_call(
        _transpose_body,
        grid=(NTB,),
        in_specs=[pl.BlockSpec((D, TBLK), lambda i: (0, i))],
        out_specs=pl.BlockSpec((TBLK // 2, DP), lambda i: (i, 0)),
        out_shape=jax.ShapeDtypeStruct((VP, DP), jnp.float32),
    )(w_t)

    mesh = plsc.VectorSubcoreMesh(core_axis_name="c", subcore_axis_name="s")
    gather = functools.partial(
        pl.kernel,
        mesh=mesh,
        compiler_params=pltpu.CompilerParams(
            needs_layout_passes=False, use_tc_tiling_on_sc=True),
        out_type=jax.ShapeDtypeStruct((B, D), jnp.float32),
        scratch_types=[
            pltpu.VMEM((L, CH), jnp.int32),
            pltpu.VMEM((L, CH), jnp.int32),
            pltpu.VMEM((NID,), jnp.int32),
            pltpu.VMEM((PW * LH,), jnp.int32),
            pltpu.VMEM((PW,), jnp.float32),
            pltpu.VMEM((RING, DP), jnp.float32),
            pltpu.VMEM((2, SPP, D), jnp.float32),
            pltpu.SemaphoreType.DMA,
            pltpu.SemaphoreType.DMA,
            pltpu.SemaphoreType.DMA,
            pltpu.SemaphoreType.DMA,
            pltpu.SemaphoreType.DMA,
        ],
    )(_qenc_body)
    return gather(ids_t, wpairs)


def kernel(seqs, W):
    return _qenc(seqs.T, W.T)


# speedup vs baseline: 1.2998x; 1.2998x over previous
import functools

import jax
import jax.numpy as jnp
from jax import lax
from jax.experimental import pallas as pl
from jax.experimental.pallas import tpu as pltpu
from jax.experimental.pallas import tpu_sc as plsc

B = 16384
L = 50
D = 64
NC = 2
NS = 16
NW = NC * NS
PW = B // NW
NID = PW * L
NLANE = 16
ND = D // NLANE
CH = 64
NCH = PW // CH
SPP = 16
RING = SPP * L
NP = PW // SPP
HALF = RING // 2
GROUP = 200
SPLITS = ((0, 104), (104, 96))


def _qenc_body(idst_hbm, w_hbm, out_hbm,
               stage0, stage1, idsf_v, inv_v, ring_v, out_v,
               semS0, semS1, semA, semB):
    wid = lax.axis_index("s") * NC + lax.axis_index("c")
    sbase = wid * PW
    lane = lax.iota(jnp.int32, NLANE)
    lane50 = lane * L

    stages = (stage0, stage1)
    sems = (semS0, semS1)

    def stage_copy(c, buf, sem):
        return pltpu.async_copy(
            idst_hbm.at[:, pl.ds(sbase + CH * c, CH)], buf, sem)

    stage_copy(0, stages[0], sems[0])
    for c in range(NCH):
        buf, sem = stages[c % 2], sems[c % 2]
        if c + 1 < NCH:
            stage_copy(c + 1, stages[(c + 1) % 2], sems[(c + 1) % 2])
        pltpu.make_async_copy(
            idst_hbm.at[:, pl.ds(sbase + CH * c, CH)], buf, sem).wait()
        for k in range(CH // NLANE):
            base50 = (CH * c + NLANE * k) * L

            def tl(l, cnt, _k=k, _base50=base50, _buf=buf):
                v = _buf[l, pl.ds(NLANE * _k, NLANE)]
                plsc.store_scatter(idsf_v, [lane50 + (_base50 + l)], v)
                return cnt + jnp.where(v != 0, 1.0, 0.0)

            cnt = lax.fori_loop(
                0, L, tl, jnp.zeros((NLANE,), jnp.float32))
            inv_v[pl.ds(CH * c + NLANE * k, NLANE)] = (
                1.0 / jnp.maximum(cnt, 1.0))

    def half_streams(p, half):
        base = pl.multiple_of(RING * p + HALF * half, 8)
        out = []
        for g in range(HALF // GROUP):
            for off, size in SPLITS:
                src = w_hbm.at[idsf_v.at[pl.ds(base + GROUP * g + off, size)]]
                dst = ring_v.at[pl.ds(HALF * half + GROUP * g + off, size)]
                out.append((src, dst))
        return out

    def issue(p, half, sem):
        for src, dst in half_streams(p, half):
            pltpu.async_copy(src, dst, sem)

    def drain(p, half, sem):
        for src, dst in half_streams(p, half):
            pltpu.make_async_copy(src, dst, sem).wait()

    def reduce_half(p, half):
        def one(j, carry):
            rb = HALF * half + L * j
            accs = [ring_v[rb, pl.ds(d * NLANE, NLANE)] for d in range(ND)]
            for l in range(1, L):
                for d in range(ND):
                    accs[d] = accs[d] + ring_v[rb + l, pl.ds(d * NLANE, NLANE)]
            s = SPP * p + 8 * half + j
            inv = plsc.load_gather(inv_v, [jnp.zeros((NLANE,), jnp.int32) + s])
            for d in range(ND):
                out_v[s, pl.ds(d * NLANE, NLANE)] = accs[d] * inv
            return carry

        lax.fori_loop(0, SPP // 2, one, 0)

    issue(0, 0, semA)
    issue(0, 1, semB)

    def body(p, carry):
        drain(p, 0, semA)
        reduce_half(p, 0)

        @pl.when(p + 1 < NP)
        def _():
            issue(p + 1, 0, semA)

        drain(p, 1, semB)
        reduce_half(p, 1)

        @pl.when(p + 1 < NP)
        def _():
            issue(p + 1, 1, semB)

        return carry

    lax.fori_loop(0, NP, body, 0)
    pltpu.sync_copy(out_v, out_hbm.at[pl.ds(sbase, PW)])


@jax.jit
def _qenc(ids_t, w):
    mesh = plsc.VectorSubcoreMesh(core_axis_name="c", subcore_axis_name="s")
    f = functools.partial(
        pl.kernel,
        mesh=mesh,
        compiler_params=pltpu.CompilerParams(
            needs_layout_passes=False, use_tc_tiling_on_sc=False),
        out_type=jax.ShapeDtypeStruct((B, D), jnp.float32),
        scratch_types=[
            pltpu.VMEM((L, CH), jnp.int32),
            pltpu.VMEM((L, CH), jnp.int32),
            pltpu.VMEM((NID,), jnp.int32),
            pltpu.VMEM((PW,), jnp.float32),
            pltpu.VMEM((RING, D), jnp.float32),
            pltpu.VMEM((PW, D), jnp.float32),
            pltpu.SemaphoreType.DMA,
            pltpu.SemaphoreType.DMA,
            pltpu.SemaphoreType.DMA,
            pltpu.SemaphoreType.DMA,
        ],
    )(_qenc_body)
    return f(ids_t, w)


def kernel(seqs, W):
    return _qenc(seqs.T, W)
